# Initial kernel scaffold; baseline (speedup 1.0000x reference)
import functools
import jax
import jax.numpy as jnp
from jax import lax
from jax.experimental import pallas as pl
from jax.experimental.pallas import tpu as pltpu
from jax.experimental.pallas import tpu_sc as plsc

_N = 10000
_E = 320000
_H = 128
_VOCAB = 28
_G = 128

_NC = 2
_NS = 16
_NW = _NC * _NS

_CH = 128
_NP = 10240
_RPS = _NP // _NS
_ECHUNKS = -(-_E // (_NW * _CH))
_EPT = _ECHUNKS * _CH
_EP = _EPT * _NW

_mesh = plsc.VectorSubcoreMesh(
    core_axis_name="c", subcore_axis_name="s", num_cores=_NC, num_subcores=_NS
)


@functools.partial(
    pl.kernel,
    out_type=jax.ShapeDtypeStruct((_NC, _NP), jnp.float32),
    mesh=_mesh,
    scratch_types=dict(
        acc=pltpu.VMEM_SHARED((_NP,), jnp.float32),
        zb=pltpu.VMEM((_CH,), jnp.float32),
        ones=pltpu.VMEM((_CH,), jnp.float32),
        didx=pltpu.VMEM((_CH,), jnp.int32),
    ),
)
def _deg_kernel(zeros1_hbm, ones1_hbm, dst_hbm, out_hbm, acc, zb, ones, didx):
    cid = lax.axis_index("c")
    sid = lax.axis_index("s")
    pltpu.sync_copy(zeros1_hbm, zb)
    pltpu.sync_copy(ones1_hbm, ones)
    r0 = sid * _RPS
    for k in range(_RPS // _CH):
        pltpu.sync_copy(zb, acc.at[pl.ds(r0 + k * _CH, _CH)])
    plsc.subcore_barrier()

    wid = sid * _NC + cid
    base = wid * _EPT

    def body(t, carry):
        e0 = base + t * _CH
        pltpu.sync_copy(dst_hbm.at[pl.ds(e0, _CH)], didx)
        pltpu.sync_copy(ones, acc.at[didx], add=True)
        return carry

    lax.fori_loop(0, _ECHUNKS, body, 0)
    plsc.subcore_barrier()
    for k in range(_RPS // _CH):
        pltpu.sync_copy(acc.at[pl.ds(r0 + k * _CH, _CH)], zb)
        pltpu.sync_copy(zb, out_hbm.at[cid, pl.ds(r0 + k * _CH, _CH)])


@functools.partial(
    pl.kernel,
    out_type=jax.ShapeDtypeStruct((_NC, _NP, _H), jnp.float32),
    mesh=_mesh,
    scratch_types=dict(
        acc=pltpu.VMEM_SHARED((_NP, _H), jnp.float32),
        zb=pltpu.VMEM((_CH, _H), jnp.float32),
        sidx=pltpu.VMEM((_CH,), jnp.int32),
        didx=pltpu.VMEM((_CH,), jnp.int32),
        rows=pltpu.VMEM((_CH, _H), jnp.float32),
        sem=pltpu.SemaphoreType.DMA,
    ),
)
def _agg_kernel(g_hbm, src_hbm, dst_hbm, zeros_hbm, out_hbm,
                acc, zb, sidx, didx, rows, sem):
    cid = lax.axis_index("c")
    sid = lax.axis_index("s")
    pltpu.sync_copy(zeros_hbm, zb)
    r0 = sid * _RPS
    for k in range(_RPS // _CH):
        pltpu.sync_copy(zb, acc.at[pl.ds(r0 + k * _CH, _CH)])
    plsc.subcore_barrier()

    wid = sid * _NC + cid
    base = wid * _EPT

    def body(t, carry):
        e0 = base + t * _CH
        pltpu.sync_copy(src_hbm.at[pl.ds(e0, _CH)], sidx)
        pltpu.sync_copy(dst_hbm.at[pl.ds(e0, _CH)], didx)
        pltpu.async_copy(g_hbm.at[sidx], rows, sem).wait()
        pltpu.sync_copy(rows, acc.at[didx], add=True)
        return carry

    lax.fori_loop(0, _ECHUNKS, body, 0)
    plsc.subcore_barrier()
    for k in range(_RPS // _CH):
        pltpu.sync_copy(acc.at[pl.ds(r0 + k * _CH, _CH)], rows)
        pltpu.sync_copy(rows, out_hbm.at[cid, pl.ds(r0 + k * _CH, _CH)])


def _tc1_body(x_ref, degp_ref, embed_ref, w1_ref, dinv_ref, g1_ref):
    indeg = degp_ref[0, :] + degp_ref[1, :]
    rows = lax.broadcasted_iota(jnp.int32, (_NP,), 0)
    dinv = jnp.where(rows < _N, lax.rsqrt(indeg + 1.0), 0.0)
    dinv_ref[...] = dinv
    oh = (x_ref[...] == lax.broadcasted_iota(jnp.int32, (_NP, _VOCAB), 1))
    embw = jnp.dot(embed_ref[...], w1_ref[...],
                   preferred_element_type=jnp.float32)
    hw = jnp.dot(oh.astype(jnp.float32), embw,
                 preferred_element_type=jnp.float32)
    g1_ref[...] = dinv[:, None] * hw


def _tc_mid_body(aggp_ref, g_ref, dinv_ref, b_ref, w_ref, gn_ref):
    dinv = dinv_ref[...]
    s = aggp_ref[0] + aggp_ref[1] + g_ref[...]
    h = jax.nn.relu(dinv[:, None] * s + b_ref[...][None, :])
    gn_ref[...] = dinv[:, None] * jnp.dot(h, w_ref[...],
                                          preferred_element_type=jnp.float32)


def _tc4_body(aggp_ref, g_ref, dinv_ref, b_ref, batch_ref,
              mw1_ref, mb1_ref, mw2_ref, mb2_ref, y_ref):
    dinv = dinv_ref[...]
    s = aggp_ref[0] + aggp_ref[1] + g_ref[...]
    h = jax.nn.relu(dinv[:, None] * s + b_ref[...][None, :])
    gid = lax.broadcasted_iota(jnp.int32, (_G, _NP), 0)
    m = (gid == batch_ref[...][None, :]).astype(jnp.float32)
    y = jnp.dot(m, h, preferred_element_type=jnp.float32)
    y = jax.nn.relu(jnp.dot(y, mw1_ref[...],
                            preferred_element_type=jnp.float32)
                    + mb1_ref[...][None, :])
    y_ref[...] = (jnp.dot(y, mw2_ref[...], preferred_element_type=jnp.float32)
                  + mb2_ref[...][None, :])


_tc1 = pl.pallas_call(
    _tc1_body,
    out_shape=(jax.ShapeDtypeStruct((_NP,), jnp.float32),
               jax.ShapeDtypeStruct((_NP, _H), jnp.float32)),
)

_tc_mid = pl.pallas_call(
    _tc_mid_body,
    out_shape=jax.ShapeDtypeStruct((_NP, _H), jnp.float32),
)

_tc4 = pl.pallas_call(
    _tc4_body,
    out_shape=jax.ShapeDtypeStruct((_G, 1), jnp.float32),
)


def kernel(x, edge_index, batch, embed, W1, b1, W2, b2, W3, b3,
           mw1, mb1, mw2, mb2):
    src = edge_index[0].astype(jnp.int32)
    dst = edge_index[1].astype(jnp.int32)
    pad = jnp.full((_EP - _E,), _N, dtype=jnp.int32)
    src_p = jnp.concatenate([src, pad])
    dst_p = jnp.concatenate([dst, pad])
    x_p = jnp.concatenate(
        [x.astype(jnp.int32).reshape(_N, 1),
         jnp.zeros((_NP - _N, 1), jnp.int32)])
    batch_p = jnp.concatenate(
        [batch.astype(jnp.int32), jnp.full((_NP - _N,), _G, dtype=jnp.int32)])
    zeros2 = jnp.zeros((_CH, _H), jnp.float32)
    zeros1 = jnp.zeros((_CH,), jnp.float32)
    ones1 = jnp.ones((_CH,), jnp.float32)

    degp = _deg_kernel(zeros1, ones1, dst_p)
    dinv, g1 = _tc1(x_p, degp, embed, W1)
    a1 = _agg_kernel(g1, src_p, dst_p, zeros2)
    g2 = _tc_mid(a1, g1, dinv, b1, W2)
    a2 = _agg_kernel(g2, src_p, dst_p, zeros2)
    g3 = _tc_mid(a2, g2, dinv, b2, W3)
    a3 = _agg_kernel(g3, src_p, dst_p, zeros2)
    y = _tc4(a3, g3, dinv, b3, batch_p, mw1, mb1, mw2, mb2)
    return y[:, 0]


# speedup vs baseline: 8.5979x; 8.5979x over previous
import functools
import jax
import jax.numpy as jnp
from jax import lax
from jax.experimental import pallas as pl
from jax.experimental.pallas import tpu as pltpu
from jax.experimental.pallas import tpu_sc as plsc

_N = 10000
_E = 320000
_H = 128
_VOCAB = 28
_G = 128

_NC = 2
_NS = 16
_NW = _NC * _NS

_CH = 128
_NP = 10240
_RPS = _NP // _NS
_ECHUNKS = -(-_E // (_NW * _CH))
_EPT = _ECHUNKS * _CH
_EP = _EPT * _NW

def _deg_body(zeros1_hbm, ones1_hbm, dst_hbm, out_hbm, acc, zb, ones, didx):
    cid = lax.axis_index("c")
    sid = lax.axis_index("s")
    pltpu.sync_copy(zeros1_hbm, zb)
    pltpu.sync_copy(ones1_hbm, ones)
    r0 = sid * _RPS
    for k in range(_RPS // _CH):
        pltpu.sync_copy(zb, acc.at[pl.ds(r0 + k * _CH, _CH)])
    plsc.subcore_barrier()

    wid = sid * _NC + cid
    base = wid * _EPT

    def body(t, carry):
        e0 = base + t * _CH
        pltpu.sync_copy(dst_hbm.at[pl.ds(e0, _CH)], didx)
        pltpu.sync_copy(ones, acc.at[didx], add=True)
        return carry

    lax.fori_loop(0, _ECHUNKS, body, 0)
    plsc.subcore_barrier()
    for k in range(_RPS // _CH):
        pltpu.sync_copy(acc.at[pl.ds(r0 + k * _CH, _CH)], zb)
        pltpu.sync_copy(zb, out_hbm.at[cid, pl.ds(r0 + k * _CH, _CH)])


def _agg_body(g_hbm, src_hbm, dst_hbm, zeros_hbm, out_hbm,
              acc, zb, sidx, didx, rows, sem):
    cid = lax.axis_index("c")
    sid = lax.axis_index("s")
    pltpu.sync_copy(zeros_hbm, zb)
    r0 = sid * _RPS
    for k in range(_RPS // _CH):
        pltpu.sync_copy(zb, acc.at[pl.ds(r0 + k * _CH, _CH)])
    plsc.subcore_barrier()

    wid = sid * _NC + cid
    base = wid * _EPT

    def body(t, carry):
        e0 = base + t * _CH
        pltpu.sync_copy(src_hbm.at[pl.ds(e0, _CH)], sidx)
        pltpu.sync_copy(dst_hbm.at[pl.ds(e0, _CH)], didx)
        pltpu.async_copy(g_hbm.at[sidx], rows, sem).wait()
        pltpu.sync_copy(rows, acc.at[didx], add=True)
        return carry

    lax.fori_loop(0, _ECHUNKS, body, 0)
    plsc.subcore_barrier()
    for k in range(_RPS // _CH):
        pltpu.sync_copy(acc.at[pl.ds(r0 + k * _CH, _CH)], rows)
        pltpu.sync_copy(rows, out_hbm.at[cid, pl.ds(r0 + k * _CH, _CH)])


@functools.cache
def _sc_kernels():
    mesh = plsc.VectorSubcoreMesh(
        core_axis_name="c", subcore_axis_name="s",
        num_cores=_NC, num_subcores=_NS)
    deg = pl.kernel(
        _deg_body,
        out_type=jax.ShapeDtypeStruct((_NC, _NP), jnp.float32),
        mesh=mesh,
        scratch_types=dict(
            acc=pltpu.VMEM_SHARED((_NP,), jnp.float32),
            zb=pltpu.VMEM((_CH,), jnp.float32),
            ones=pltpu.VMEM((_CH,), jnp.float32),
            didx=pltpu.VMEM((_CH,), jnp.int32),
        ),
    )
    agg = pl.kernel(
        _agg_body,
        out_type=jax.ShapeDtypeStruct((_NC, _NP, _H), jnp.float32),
        mesh=mesh,
        scratch_types=dict(
            acc=pltpu.VMEM_SHARED((_NP, _H), jnp.float32),
            zb=pltpu.VMEM((_CH, _H), jnp.float32),
            sidx=pltpu.VMEM((_CH,), jnp.int32),
            didx=pltpu.VMEM((_CH,), jnp.int32),
            rows=pltpu.VMEM((_CH, _H), jnp.float32),
            sem=pltpu.SemaphoreType.DMA,
        ),
    )
    return deg, agg


def _tc1_body(x_ref, degp_ref, embed_ref, w1_ref, dinv_ref, g1_ref):
    indeg = degp_ref[0, :] + degp_ref[1, :]
    rows = lax.broadcasted_iota(jnp.int32, (_NP,), 0)
    dinv = jnp.where(rows < _N, lax.rsqrt(indeg + 1.0), 0.0)
    dinv_ref[...] = dinv
    oh = (x_ref[...] == lax.broadcasted_iota(jnp.int32, (_NP, _VOCAB), 1))
    embw = jnp.dot(embed_ref[...], w1_ref[...],
                   preferred_element_type=jnp.float32)
    hw = jnp.dot(oh.astype(jnp.float32), embw,
                 preferred_element_type=jnp.float32,
                 precision=lax.Precision.HIGHEST)
    g1_ref[...] = dinv[:, None] * hw


def _tc_mid_body(aggp_ref, g_ref, dinv_ref, b_ref, w_ref, gn_ref):
    dinv = dinv_ref[...]
    s = aggp_ref[0] + aggp_ref[1] + g_ref[...]
    h = jax.nn.relu(dinv[:, None] * s + b_ref[...][None, :])
    gn_ref[...] = dinv[:, None] * jnp.dot(h, w_ref[...],
                                          preferred_element_type=jnp.float32)


def _tc4_body(aggp_ref, g_ref, dinv_ref, b_ref, batch_ref,
              mw1_ref, mb1_ref, mw2_ref, mb2_ref, y_ref):
    dinv = dinv_ref[...]
    s = aggp_ref[0] + aggp_ref[1] + g_ref[...]
    h = jax.nn.relu(dinv[:, None] * s + b_ref[...][None, :])
    gid = lax.broadcasted_iota(jnp.int32, (_G, _NP), 0)
    m = (gid == batch_ref[...][None, :]).astype(jnp.float32)
    y = jnp.dot(m, h, preferred_element_type=jnp.float32,
                precision=lax.Precision.HIGHEST)
    y = jax.nn.relu(jnp.dot(y, mw1_ref[...],
                            preferred_element_type=jnp.float32)
                    + mb1_ref[...][None, :])
    y_ref[...] = (jnp.dot(y, mw2_ref[...], preferred_element_type=jnp.float32)
                  + mb2_ref[...][None, :])


_tc1 = pl.pallas_call(
    _tc1_body,
    out_shape=(jax.ShapeDtypeStruct((_NP,), jnp.float32),
               jax.ShapeDtypeStruct((_NP, _H), jnp.float32)),
)

_tc_mid = pl.pallas_call(
    _tc_mid_body,
    out_shape=jax.ShapeDtypeStruct((_NP, _H), jnp.float32),
)

_tc4 = pl.pallas_call(
    _tc4_body,
    out_shape=jax.ShapeDtypeStruct((_G, 1), jnp.float32),
)


def kernel(x, edge_index, batch, embed, W1, b1, W2, b2, W3, b3,
           mw1, mb1, mw2, mb2):
    src = edge_index[0].astype(jnp.int32)
    dst = edge_index[1].astype(jnp.int32)
    pad = jnp.full((_EP - _E,), _N, dtype=jnp.int32)
    src_p = jnp.concatenate([src, pad])
    dst_p = jnp.concatenate([dst, pad])
    x_p = jnp.concatenate(
        [x.astype(jnp.int32).reshape(_N, 1),
         jnp.zeros((_NP - _N, 1), jnp.int32)])
    batch_p = jnp.concatenate(
        [batch.astype(jnp.int32), jnp.full((_NP - _N,), _G, dtype=jnp.int32)])
    zeros2 = jnp.zeros((_CH, _H), jnp.float32)
    zeros1 = jnp.zeros((_CH,), jnp.float32)
    ones1 = jnp.ones((_CH,), jnp.float32)

    _deg_kernel, _agg_kernel = _sc_kernels()
    degp = _deg_kernel(zeros1, ones1, dst_p)
    dinv, g1 = _tc1(x_p, degp, embed, W1)
    a1 = _agg_kernel(g1, src_p, dst_p, zeros2)
    g2 = _tc_mid(a1, g1, dinv, b1, W2)
    a2 = _agg_kernel(g2, src_p, dst_p, zeros2)
    g3 = _tc_mid(a2, g2, dinv, b2, W3)
    a3 = _agg_kernel(g3, src_p, dst_p, zeros2)
    y = _tc4(a3, g3, dinv, b3, batch_p, mw1, mb1, mw2, mb2)
    return y[:, 0]
